# Initial kernel scaffold; baseline (speedup 1.0000x reference)
import jax
import jax.numpy as jnp
from jax.experimental import pallas as pl


def kernel(x, W_in, conv_w, conv_b, dt_bias, A_log, D_param, norm_w, W_mamba_out, idx_q_w, idx_k_w, q_down_w, q_up_w, q_rope_w, kv_down_w, kv_up_w, k_rope_w, out_w):
    raise NotImplementedError("write your pallas kernel here")


# speedup vs baseline: 17.1319x; 17.1319x over previous
import jax
import jax.numpy as jnp
from jax.experimental import pallas as pl
from jax.experimental.pallas import tpu as pltpu

SEQ = 2048
D_MODEL = 1024
D_STATE = 64
HEADDIM = 32
D_INNER = 1024
NHEADS_M = D_INNER // HEADDIM
CONV_K = 4
N_HEADS = 12
V_HEAD = 64
ROPE = 32
IDX_DIM = 64
TOP_K = 64

MAMBA_CHUNK = 128
PROJ_BLK = 512
ATTN_BLK = 256

_INTERP = False

NEG = float("-inf")


def _mm16(a, b, dims):
    return jax.lax.dot_general(
        a.astype(jnp.bfloat16), b.astype(jnp.bfloat16), (dims, ((), ())),
        preferred_element_type=jnp.float32)


def _mmd(a, b, dims):
    return jax.lax.dot_general(
        a, b, (dims, ((), ())), preferred_element_type=jnp.float32)


def _mamba_body(x_ref, w_in_ref, conv_w_ref, conv_b_ref, dt_bias_ref,
                a_log_ref, d_param_ref, norm_w_ref, w_out_ref,
                y_out_ref, h_ref, tail_ref, da_ref, xdt_ref, bc_ref, ys_ref):
    i = pl.program_id(0)

    @pl.when(i == 0)
    def _init():
        h_ref[...] = jnp.zeros_like(h_ref)
        tail_ref[...] = jnp.zeros_like(tail_ref)

    xc = x_ref[...]
    zxbcdt = _mmd(xc, w_in_ref[...], ((1,), (1,)))
    z = zxbcdt[:, :D_INNER]
    xbc_raw = zxbcdt[:, D_INNER:D_INNER + D_INNER + 2 * D_STATE]
    dt_raw = zxbcdt[:, -NHEADS_M:]

    padded = jnp.concatenate([tail_ref[...], xbc_raw], axis=0)
    acc = padded[0:MAMBA_CHUNK, :] * conv_w_ref[:, 0][None, :]
    for k in range(1, CONV_K):
        acc = acc + padded[k:k + MAMBA_CHUNK, :] * conv_w_ref[:, k][None, :]
    acc = acc + conv_b_ref[...]
    tail_ref[...] = xbc_raw[MAMBA_CHUNK - (CONV_K - 1):, :]
    xbc = acc * jax.nn.sigmoid(acc)

    bmat = xbc[:, D_INNER:D_INNER + D_STATE]
    cmat = xbc[:, D_INNER + D_STATE:]
    dt = jax.nn.softplus(dt_raw + dt_bias_ref[...])
    a_neg = -jnp.exp(a_log_ref[...])
    dloga = dt * a_neg

    q = MAMBA_CHUNK
    dA = jnp.exp(dloga)

    da_parts = []
    xdt_parts = []
    dxs_parts = []
    for h in range(NHEADS_M):
        xs_h = xbc[:, HEADDIM * h:HEADDIM * (h + 1)]
        da_parts.append(jnp.broadcast_to(dA[:, h:h + 1], (q, HEADDIM)))
        xdt_parts.append(dt[:, h:h + 1] * xs_h)
        dxs_parts.append(d_param_ref[0, h] * xs_h)
    da_ref[...] = jnp.concatenate(da_parts, axis=1)
    xdt_ref[...] = jnp.concatenate(xdt_parts, axis=1)
    dxs = jnp.concatenate(dxs_parts, axis=1)
    bc_ref[...] = jnp.concatenate([bmat, cmat], axis=1)

    def step(t, _):
        da_row = da_ref[pl.ds(t, 1), :]
        xdt_row = xdt_ref[pl.ds(t, 1), :]
        bc_row = bc_ref[pl.ds(t, 1), :]
        b_col = jnp.transpose(bc_row[:, :D_STATE])
        hst = h_ref[...] * da_row + xdt_row * b_col
        h_ref[...] = hst
        y_row = jax.lax.dot_general(
            bc_row[:, D_STATE:].astype(jnp.bfloat16),
            hst.astype(jnp.bfloat16),
            (((1,), (0,)), ((), ())),
            preferred_element_type=jnp.float32)
        ys_ref[pl.ds(t, 1), :] = y_row
        return 0

    jax.lax.fori_loop(0, q, step, 0, unroll=2)

    y = ys_ref[...] + dxs
    y = y * (z * jax.nn.sigmoid(z))
    y = y * jax.lax.rsqrt(jnp.mean(y * y, axis=-1, keepdims=True) + 1e-6)
    y = y * norm_w_ref[...]
    y_out_ref[...] = _mmd(y, w_out_ref[...], ((1,), (1,)))


def _rope2d(x, sin, cos):
    half = x.shape[-1] // 2
    rot = jnp.concatenate([-x[:, half:], x[:, :half]], axis=1)
    return x * cos + rot * sin


def _proj_body(x_ref, xm_ref, idx_q_ref, idx_k_ref, q_down_ref, q_up_ref,
               q_rope_ref, kv_down_ref, kv_up_ref, k_rope_ref, sin_ref, cos_ref,
               q_idx_out, k_idx_out, q_final_out, k_final_out, v_out):
    x = x_ref[...]
    xm = xm_ref[...]
    sin = sin_ref[...]
    cos = cos_ref[...]

    def mm_t(a, w):
        return _mm16(a, w, ((1,), (1,)))

    q_idx_out[...] = _mmd(xm, idx_q_ref[...], ((1,), (1,)))
    k_idx_out[...] = _mmd(x, idx_k_ref[...], ((1,), (1,)))

    c_q = mm_t(x, q_down_ref[...])
    q_content = mm_t(c_q, q_up_ref[...])
    q_rope = mm_t(c_q, q_rope_ref[...])
    c_kv = mm_t(x, kv_down_ref[...])
    kv = mm_t(c_kv, kv_up_ref[...])
    k_rope = _rope2d(mm_t(x, k_rope_ref[...]), sin, cos)

    q_parts = []
    k_parts = []
    for h in range(N_HEADS):
        qr_h = _rope2d(q_rope[:, ROPE * h:ROPE * (h + 1)], sin, cos)
        q_parts.append(q_content[:, V_HEAD * h:V_HEAD * (h + 1)])
        q_parts.append(qr_h)
        k_parts.append(kv[:, V_HEAD * h:V_HEAD * (h + 1)])
        k_parts.append(k_rope)
    q_final_out[...] = jnp.concatenate(q_parts, axis=1)
    k_final_out[...] = jnp.concatenate(k_parts, axis=1)
    v_out[...] = kv[:, N_HEADS * V_HEAD:]


def _attn_body(q_idx_ref, k_idx_ref, q_final_ref, k_final_ref, v_ref,
               out_w_ref, out_ref):
    b = pl.program_id(0)
    m = ATTN_BLK
    rowi = jax.lax.broadcasted_iota(jnp.int32, (m, SEQ), 0) + b * m
    coli = jax.lax.broadcasted_iota(jnp.int32, (m, SEQ), 1)
    causal = rowi >= coli

    s_idx = _mmd(q_idx_ref[...], k_idx_ref[...],
                 ((1,), (1,))) * (IDX_DIM ** -0.5)
    w0 = jnp.where(causal, s_idx, NEG)

    def ext_body(_, w):
        mx = jnp.max(w, axis=1, keepdims=True)
        return jnp.where(w == mx, NEG, w)

    w_fin = jax.lax.fori_loop(0, TOP_K, ext_body, w0)
    sel = ((rowi < TOP_K) & (coli < TOP_K)) | ((rowi >= TOP_K) & (w_fin != w0))

    scale = (V_HEAD + ROPE) ** -0.5
    head_outs = []
    for h in range(N_HEADS):
        hd = V_HEAD + ROPE
        q_h = q_final_ref[:, hd * h:hd * (h + 1)]
        k_h = k_final_ref[:, hd * h:hd * (h + 1)]
        v_h = v_ref[:, V_HEAD * h:V_HEAD * (h + 1)]
        logits = _mm16(q_h, k_h, ((1,), (1,))) * scale
        logits = jnp.where(sel, logits, NEG)
        mx = jnp.max(logits, axis=1, keepdims=True)
        p = jnp.exp(logits - mx)
        denom = jnp.sum(p, axis=1, keepdims=True)
        o_h = _mm16(p, v_h, ((1,), (0,)))
        head_outs.append(o_h / denom)
    attn = jnp.concatenate(head_outs, axis=1)
    out_ref[...] = _mm16(attn, out_w_ref[...], ((1,), (1,)))


def _stage1(x, W_in, conv_w, conv_b, dt_bias, A_log, D_param, norm_w,
            W_mamba_out):
    seq = x.shape[0]
    conv_dim = D_INNER + 2 * D_STATE
    d_in_proj = 2 * D_INNER + 2 * D_STATE + NHEADS_M

    conv_b2 = conv_b.reshape(1, conv_dim)
    dt_bias2 = dt_bias.reshape(1, NHEADS_M)
    a_log2 = A_log.reshape(1, NHEADS_M)
    d_param2 = D_param.reshape(1, NHEADS_M)
    norm_w2 = norm_w.reshape(1, D_INNER)

    n_chunks = seq // MAMBA_CHUNK
    x_mamba = pl.pallas_call(
        _mamba_body,
        grid=(n_chunks,),
        in_specs=[
            pl.BlockSpec((MAMBA_CHUNK, D_MODEL), lambda i: (i, 0)),
            pl.BlockSpec((d_in_proj, D_MODEL), lambda i: (0, 0)),
            pl.BlockSpec((conv_dim, CONV_K), lambda i: (0, 0)),
            pl.BlockSpec((1, conv_dim), lambda i: (0, 0)),
            pl.BlockSpec((1, NHEADS_M), lambda i: (0, 0)),
            pl.BlockSpec((1, NHEADS_M), lambda i: (0, 0)),
            pl.BlockSpec((1, NHEADS_M), lambda i: (0, 0)),
            pl.BlockSpec((1, D_INNER), lambda i: (0, 0)),
            pl.BlockSpec((D_MODEL, D_INNER), lambda i: (0, 0)),
        ],
        out_specs=pl.BlockSpec((MAMBA_CHUNK, D_MODEL), lambda i: (i, 0)),
        out_shape=jax.ShapeDtypeStruct((seq, D_MODEL), jnp.float32),
        scratch_shapes=[
            pltpu.VMEM((D_STATE, D_INNER), jnp.float32),
            pltpu.VMEM((CONV_K - 1, conv_dim), jnp.float32),
            pltpu.VMEM((MAMBA_CHUNK, D_INNER), jnp.float32),
            pltpu.VMEM((MAMBA_CHUNK, D_INNER), jnp.float32),
            pltpu.VMEM((MAMBA_CHUNK, 2 * D_STATE), jnp.float32),
            pltpu.VMEM((MAMBA_CHUNK, D_INNER), jnp.float32),
        ],
        interpret=_INTERP,
    )(x, W_in, conv_w, conv_b2, dt_bias2, a_log2, d_param2, norm_w2,
      W_mamba_out)
    return x_mamba


def _stage2(x, x_mamba, idx_q_w, idx_k_w, q_down_w, q_up_w, q_rope_w,
            kv_down_w, kv_up_w, k_rope_w):
    seq = x.shape[0]
    inv_freq = 1.0 / (10000.0 ** (jnp.arange(0, ROPE, 2, dtype=jnp.float32)
                                  / ROPE))
    t = jnp.arange(seq, dtype=jnp.float32)
    freqs = jnp.outer(t, inv_freq)
    emb = jnp.concatenate([freqs, freqs], axis=-1)
    sin, cos = jnp.sin(emb), jnp.cos(emb)

    n_pb = seq // PROJ_BLK
    q_idx, k_idx, q_final, k_final, v = pl.pallas_call(
        _proj_body,
        grid=(n_pb,),
        in_specs=[
            pl.BlockSpec((PROJ_BLK, D_MODEL), lambda i: (i, 0)),
            pl.BlockSpec((PROJ_BLK, D_MODEL), lambda i: (i, 0)),
            pl.BlockSpec((IDX_DIM, D_MODEL), lambda i: (0, 0)),
            pl.BlockSpec((IDX_DIM, D_MODEL), lambda i: (0, 0)),
            pl.BlockSpec((128, D_MODEL), lambda i: (0, 0)),
            pl.BlockSpec((N_HEADS * V_HEAD, 128), lambda i: (0, 0)),
            pl.BlockSpec((N_HEADS * ROPE, 128), lambda i: (0, 0)),
            pl.BlockSpec((128, D_MODEL), lambda i: (0, 0)),
            pl.BlockSpec((2 * N_HEADS * V_HEAD, 128), lambda i: (0, 0)),
            pl.BlockSpec((ROPE, D_MODEL), lambda i: (0, 0)),
            pl.BlockSpec((PROJ_BLK, ROPE), lambda i: (i, 0)),
            pl.BlockSpec((PROJ_BLK, ROPE), lambda i: (i, 0)),
        ],
        out_specs=[
            pl.BlockSpec((PROJ_BLK, IDX_DIM), lambda i: (i, 0)),
            pl.BlockSpec((PROJ_BLK, IDX_DIM), lambda i: (i, 0)),
            pl.BlockSpec((PROJ_BLK, N_HEADS * (V_HEAD + ROPE)),
                         lambda i: (i, 0)),
            pl.BlockSpec((PROJ_BLK, N_HEADS * (V_HEAD + ROPE)),
                         lambda i: (i, 0)),
            pl.BlockSpec((PROJ_BLK, N_HEADS * V_HEAD), lambda i: (i, 0)),
        ],
        out_shape=[
            jax.ShapeDtypeStruct((seq, IDX_DIM), jnp.float32),
            jax.ShapeDtypeStruct((seq, IDX_DIM), jnp.float32),
            jax.ShapeDtypeStruct((seq, N_HEADS * (V_HEAD + ROPE)), jnp.float32),
            jax.ShapeDtypeStruct((seq, N_HEADS * (V_HEAD + ROPE)), jnp.float32),
            jax.ShapeDtypeStruct((seq, N_HEADS * V_HEAD), jnp.float32),
        ],
        interpret=_INTERP,
    )(x, x_mamba, idx_q_w, idx_k_w, q_down_w, q_up_w, q_rope_w,
      kv_down_w, kv_up_w, k_rope_w, sin, cos)
    return q_idx, k_idx, q_final, k_final, v


def _stage3(q_idx, k_idx, q_final, k_final, v, out_w):
    seq = q_idx.shape[0]
    n_ab = seq // ATTN_BLK
    out = pl.pallas_call(
        _attn_body,
        grid=(n_ab,),
        in_specs=[
            pl.BlockSpec((ATTN_BLK, IDX_DIM), lambda i: (i, 0)),
            pl.BlockSpec((seq, IDX_DIM), lambda i: (0, 0)),
            pl.BlockSpec((ATTN_BLK, N_HEADS * (V_HEAD + ROPE)),
                         lambda i: (i, 0)),
            pl.BlockSpec((seq, N_HEADS * (V_HEAD + ROPE)), lambda i: (0, 0)),
            pl.BlockSpec((seq, N_HEADS * V_HEAD), lambda i: (0, 0)),
            pl.BlockSpec((D_MODEL, N_HEADS * V_HEAD), lambda i: (0, 0)),
        ],
        out_specs=pl.BlockSpec((ATTN_BLK, D_MODEL), lambda i: (i, 0)),
        out_shape=jax.ShapeDtypeStruct((seq, D_MODEL), jnp.float32),
        interpret=_INTERP,
    )(q_idx, k_idx, q_final, k_final, v, out_w)
    return out


def kernel(x, W_in, conv_w, conv_b, dt_bias, A_log, D_param, norm_w,
           W_mamba_out, idx_q_w, idx_k_w, q_down_w, q_up_w, q_rope_w,
           kv_down_w, kv_up_w, k_rope_w, out_w):
    x_mamba = _stage1(x, W_in, conv_w, conv_b, dt_bias, A_log, D_param,
                      norm_w, W_mamba_out)
    q_idx, k_idx, q_final, k_final, v = _stage2(
        x, x_mamba, idx_q_w, idx_k_w, q_down_w, q_up_w, q_rope_w,
        kv_down_w, kv_up_w, k_rope_w)
    return _stage3(q_idx, k_idx, q_final, k_final, v, out_w)


# speedup vs baseline: 19.5685x; 1.1422x over previous
import jax
import jax.numpy as jnp
from jax.experimental import pallas as pl
from jax.experimental.pallas import tpu as pltpu

SEQ = 2048
D_MODEL = 1024
D_STATE = 64
HEADDIM = 32
D_INNER = 1024
NHEADS_M = D_INNER // HEADDIM
CONV_K = 4
N_HEADS = 12
V_HEAD = 64
ROPE = 32
IDX_DIM = 64
TOP_K = 64

MAMBA_CHUNK = 128
PROJ_BLK = 512
ATTN_BLK = 256

NEG = float("-inf")


def _mm16(a, b, dims):
    return jax.lax.dot_general(
        a.astype(jnp.bfloat16), b.astype(jnp.bfloat16), (dims, ((), ())),
        preferred_element_type=jnp.float32)


def _mmd(a, b, dims):
    return jax.lax.dot_general(
        a, b, (dims, ((), ())), preferred_element_type=jnp.float32)


def _mamba_body(x_ref, w_in_ref, conv_w_ref, conv_b_ref, dt_bias_ref,
                a_log_ref, d_param_ref, norm_w_ref, w_out_ref,
                y_out_ref, h_ref, tail_ref, da_ref, xdt_ref, bc_ref, ys_ref):
    i = pl.program_id(0)

    @pl.when(i == 0)
    def _init():
        h_ref[...] = jnp.zeros_like(h_ref)
        tail_ref[...] = jnp.zeros_like(tail_ref)

    xc = x_ref[...]
    zxbcdt = _mmd(xc, w_in_ref[...], ((1,), (1,)))
    z = zxbcdt[:, :D_INNER]
    xbc_raw = zxbcdt[:, D_INNER:D_INNER + D_INNER + 2 * D_STATE]
    dt_raw = zxbcdt[:, -NHEADS_M:]

    padded = jnp.concatenate([tail_ref[...], xbc_raw], axis=0)
    acc = padded[0:MAMBA_CHUNK, :] * conv_w_ref[:, 0][None, :]
    for k in range(1, CONV_K):
        acc = acc + padded[k:k + MAMBA_CHUNK, :] * conv_w_ref[:, k][None, :]
    acc = acc + conv_b_ref[...]
    tail_ref[...] = xbc_raw[MAMBA_CHUNK - (CONV_K - 1):, :]
    xbc = acc * jax.nn.sigmoid(acc)

    bmat = xbc[:, D_INNER:D_INNER + D_STATE]
    cmat = xbc[:, D_INNER + D_STATE:]
    dt = jax.nn.softplus(dt_raw + dt_bias_ref[...])
    a_neg = -jnp.exp(a_log_ref[...])
    dloga = dt * a_neg

    q = MAMBA_CHUNK
    dA = jnp.exp(dloga)

    da_parts = []
    xdt_parts = []
    dxs_parts = []
    for h in range(NHEADS_M):
        xs_h = xbc[:, HEADDIM * h:HEADDIM * (h + 1)]
        da_parts.append(jnp.broadcast_to(dA[:, h:h + 1], (q, HEADDIM)))
        xdt_parts.append(dt[:, h:h + 1] * xs_h)
        dxs_parts.append(d_param_ref[0, h] * xs_h)
    da_ref[...] = jnp.concatenate(da_parts, axis=1)
    xdt_ref[...] = jnp.concatenate(xdt_parts, axis=1)
    dxs = jnp.concatenate(dxs_parts, axis=1)
    bc_ref[...] = jnp.concatenate([bmat, cmat], axis=1)

    def step(t, _):
        da_row = da_ref[pl.ds(t, 1), :]
        xdt_row = xdt_ref[pl.ds(t, 1), :]
        bc_row = bc_ref[pl.ds(t, 1), :]
        b_col = jnp.transpose(bc_row[:, :D_STATE])
        hst = h_ref[...] * da_row + xdt_row * b_col
        h_ref[...] = hst
        y_row = jax.lax.dot_general(
            bc_row[:, D_STATE:].astype(jnp.bfloat16),
            hst.astype(jnp.bfloat16),
            (((1,), (0,)), ((), ())),
            preferred_element_type=jnp.float32)
        ys_ref[pl.ds(t, 1), :] = y_row
        return 0

    jax.lax.fori_loop(0, q, step, 0, unroll=4)

    y = ys_ref[...] + dxs
    y = y * (z * jax.nn.sigmoid(z))
    y = y * jax.lax.rsqrt(jnp.mean(y * y, axis=-1, keepdims=True) + 1e-6)
    y = y * norm_w_ref[...]
    y_out_ref[...] = _mmd(y, w_out_ref[...], ((1,), (1,)))


def _rope2d(x, sin, cos):
    half = x.shape[-1] // 2
    rot = jnp.concatenate([-x[:, half:], x[:, :half]], axis=1)
    return x * cos + rot * sin


def _proj_body(x_ref, xm_ref, idx_q_ref, idx_k_ref, q_down_ref, q_up_ref,
               q_rope_ref, kv_down_ref, kv_up_ref, k_rope_ref, sin_ref, cos_ref,
               q_idx_out, k_idx_out, q_final_out, k_final_out, v_out):
    x = x_ref[...]
    xm = xm_ref[...]
    sin = sin_ref[...]
    cos = cos_ref[...]

    def mm_t(a, w):
        return _mm16(a, w, ((1,), (1,)))

    q_idx_out[...] = _mmd(xm, idx_q_ref[...], ((1,), (1,)))
    k_idx_out[...] = _mmd(x, idx_k_ref[...], ((1,), (1,)))

    c_q = mm_t(x, q_down_ref[...])
    q_content = mm_t(c_q, q_up_ref[...])
    q_rope = mm_t(c_q, q_rope_ref[...])
    c_kv = mm_t(x, kv_down_ref[...])
    kv = mm_t(c_kv, kv_up_ref[...])
    k_rope = _rope2d(mm_t(x, k_rope_ref[...]), sin, cos)

    q_parts = []
    k_parts = []
    for h in range(N_HEADS):
        qr_h = _rope2d(q_rope[:, ROPE * h:ROPE * (h + 1)], sin, cos)
        q_parts.append(q_content[:, V_HEAD * h:V_HEAD * (h + 1)])
        q_parts.append(qr_h)
        k_parts.append(kv[:, V_HEAD * h:V_HEAD * (h + 1)])
        k_parts.append(k_rope)
    q_final_out[...] = jnp.concatenate(q_parts, axis=1)
    k_final_out[...] = jnp.concatenate(k_parts, axis=1)
    v_out[...] = kv[:, N_HEADS * V_HEAD:]


def _attn_body(q_idx_ref, k_idx_ref, q_final_ref, k_final_ref, v_ref,
               out_w_ref, out_ref):
    b = pl.program_id(0)
    m = ATTN_BLK
    rowi = jax.lax.broadcasted_iota(jnp.int32, (m, SEQ), 0) + b * m
    coli = jax.lax.broadcasted_iota(jnp.int32, (m, SEQ), 1)
    causal = rowi >= coli

    s_idx = _mmd(q_idx_ref[...], k_idx_ref[...],
                 ((1,), (1,))) * (IDX_DIM ** -0.5)
    w0 = jnp.where(causal, s_idx, NEG)

    def ext_body(_, w):
        mx = jnp.max(w, axis=1, keepdims=True)
        return jnp.where(w == mx, NEG, w)

    w_fin = jax.lax.fori_loop(0, TOP_K, ext_body, w0)
    sel = ((rowi < TOP_K) & (coli < TOP_K)) | ((rowi >= TOP_K) & (w_fin != w0))

    scale = (V_HEAD + ROPE) ** -0.5
    head_outs = []
    for h in range(N_HEADS):
        hd = V_HEAD + ROPE
        q_h = q_final_ref[:, hd * h:hd * (h + 1)]
        k_h = k_final_ref[:, hd * h:hd * (h + 1)]
        v_h = v_ref[:, V_HEAD * h:V_HEAD * (h + 1)]
        logits = _mm16(q_h, k_h, ((1,), (1,))) * scale
        logits = jnp.where(sel, logits, NEG)
        mx = jnp.max(logits, axis=1, keepdims=True)
        p = jnp.exp(logits - mx)
        denom = jnp.sum(p, axis=1, keepdims=True)
        o_h = _mm16(p, v_h, ((1,), (0,)))
        head_outs.append(o_h / denom)
    attn = jnp.concatenate(head_outs, axis=1)
    out_ref[...] = _mm16(attn, out_w_ref[...], ((1,), (1,)))


def _stage1(x, W_in, conv_w, conv_b, dt_bias, A_log, D_param, norm_w,
            W_mamba_out):
    seq = x.shape[0]
    conv_dim = D_INNER + 2 * D_STATE
    d_in_proj = 2 * D_INNER + 2 * D_STATE + NHEADS_M

    conv_b2 = conv_b.reshape(1, conv_dim)
    dt_bias2 = dt_bias.reshape(1, NHEADS_M)
    a_log2 = A_log.reshape(1, NHEADS_M)
    d_param2 = D_param.reshape(1, NHEADS_M)
    norm_w2 = norm_w.reshape(1, D_INNER)

    n_chunks = seq // MAMBA_CHUNK
    x_mamba = pl.pallas_call(
        _mamba_body,
        grid=(n_chunks,),
        in_specs=[
            pl.BlockSpec((MAMBA_CHUNK, D_MODEL), lambda i: (i, 0)),
            pl.BlockSpec((d_in_proj, D_MODEL), lambda i: (0, 0)),
            pl.BlockSpec((conv_dim, CONV_K), lambda i: (0, 0)),
            pl.BlockSpec((1, conv_dim), lambda i: (0, 0)),
            pl.BlockSpec((1, NHEADS_M), lambda i: (0, 0)),
            pl.BlockSpec((1, NHEADS_M), lambda i: (0, 0)),
            pl.BlockSpec((1, NHEADS_M), lambda i: (0, 0)),
            pl.BlockSpec((1, D_INNER), lambda i: (0, 0)),
            pl.BlockSpec((D_MODEL, D_INNER), lambda i: (0, 0)),
        ],
        out_specs=pl.BlockSpec((MAMBA_CHUNK, D_MODEL), lambda i: (i, 0)),
        out_shape=jax.ShapeDtypeStruct((seq, D_MODEL), jnp.float32),
        scratch_shapes=[
            pltpu.VMEM((D_STATE, D_INNER), jnp.float32),
            pltpu.VMEM((CONV_K - 1, conv_dim), jnp.float32),
            pltpu.VMEM((MAMBA_CHUNK, D_INNER), jnp.float32),
            pltpu.VMEM((MAMBA_CHUNK, D_INNER), jnp.float32),
            pltpu.VMEM((MAMBA_CHUNK, 2 * D_STATE), jnp.float32),
            pltpu.VMEM((MAMBA_CHUNK, D_INNER), jnp.float32),
        ],
    )(x, W_in, conv_w, conv_b2, dt_bias2, a_log2, d_param2, norm_w2,
      W_mamba_out)
    return x_mamba


def _stage2(x, x_mamba, idx_q_w, idx_k_w, q_down_w, q_up_w, q_rope_w,
            kv_down_w, kv_up_w, k_rope_w):
    seq = x.shape[0]
    inv_freq = 1.0 / (10000.0 ** (jnp.arange(0, ROPE, 2, dtype=jnp.float32)
                                  / ROPE))
    t = jnp.arange(seq, dtype=jnp.float32)
    freqs = jnp.outer(t, inv_freq)
    emb = jnp.concatenate([freqs, freqs], axis=-1)
    sin, cos = jnp.sin(emb), jnp.cos(emb)

    n_pb = seq // PROJ_BLK
    q_idx, k_idx, q_final, k_final, v = pl.pallas_call(
        _proj_body,
        grid=(n_pb,),
        in_specs=[
            pl.BlockSpec((PROJ_BLK, D_MODEL), lambda i: (i, 0)),
            pl.BlockSpec((PROJ_BLK, D_MODEL), lambda i: (i, 0)),
            pl.BlockSpec((IDX_DIM, D_MODEL), lambda i: (0, 0)),
            pl.BlockSpec((IDX_DIM, D_MODEL), lambda i: (0, 0)),
            pl.BlockSpec((128, D_MODEL), lambda i: (0, 0)),
            pl.BlockSpec((N_HEADS * V_HEAD, 128), lambda i: (0, 0)),
            pl.BlockSpec((N_HEADS * ROPE, 128), lambda i: (0, 0)),
            pl.BlockSpec((128, D_MODEL), lambda i: (0, 0)),
            pl.BlockSpec((2 * N_HEADS * V_HEAD, 128), lambda i: (0, 0)),
            pl.BlockSpec((ROPE, D_MODEL), lambda i: (0, 0)),
            pl.BlockSpec((PROJ_BLK, ROPE), lambda i: (i, 0)),
            pl.BlockSpec((PROJ_BLK, ROPE), lambda i: (i, 0)),
        ],
        out_specs=[
            pl.BlockSpec((PROJ_BLK, IDX_DIM), lambda i: (i, 0)),
            pl.BlockSpec((PROJ_BLK, IDX_DIM), lambda i: (i, 0)),
            pl.BlockSpec((PROJ_BLK, N_HEADS * (V_HEAD + ROPE)),
                         lambda i: (i, 0)),
            pl.BlockSpec((PROJ_BLK, N_HEADS * (V_HEAD + ROPE)),
                         lambda i: (i, 0)),
            pl.BlockSpec((PROJ_BLK, N_HEADS * V_HEAD), lambda i: (i, 0)),
        ],
        out_shape=[
            jax.ShapeDtypeStruct((seq, IDX_DIM), jnp.float32),
            jax.ShapeDtypeStruct((seq, IDX_DIM), jnp.float32),
            jax.ShapeDtypeStruct((seq, N_HEADS * (V_HEAD + ROPE)), jnp.float32),
            jax.ShapeDtypeStruct((seq, N_HEADS * (V_HEAD + ROPE)), jnp.float32),
            jax.ShapeDtypeStruct((seq, N_HEADS * V_HEAD), jnp.float32),
        ],
    )(x, x_mamba, idx_q_w, idx_k_w, q_down_w, q_up_w, q_rope_w,
      kv_down_w, kv_up_w, k_rope_w, sin, cos)
    return q_idx, k_idx, q_final, k_final, v


def _stage3(q_idx, k_idx, q_final, k_final, v, out_w):
    seq = q_idx.shape[0]
    n_ab = seq // ATTN_BLK
    out = pl.pallas_call(
        _attn_body,
        grid=(n_ab,),
        in_specs=[
            pl.BlockSpec((ATTN_BLK, IDX_DIM), lambda i: (i, 0)),
            pl.BlockSpec((seq, IDX_DIM), lambda i: (0, 0)),
            pl.BlockSpec((ATTN_BLK, N_HEADS * (V_HEAD + ROPE)),
                         lambda i: (i, 0)),
            pl.BlockSpec((seq, N_HEADS * (V_HEAD + ROPE)), lambda i: (0, 0)),
            pl.BlockSpec((seq, N_HEADS * V_HEAD), lambda i: (0, 0)),
            pl.BlockSpec((D_MODEL, N_HEADS * V_HEAD), lambda i: (0, 0)),
        ],
        out_specs=pl.BlockSpec((ATTN_BLK, D_MODEL), lambda i: (i, 0)),
        out_shape=jax.ShapeDtypeStruct((seq, D_MODEL), jnp.float32),
    )(q_idx, k_idx, q_final, k_final, v, out_w)
    return out


def kernel(x, W_in, conv_w, conv_b, dt_bias, A_log, D_param, norm_w,
           W_mamba_out, idx_q_w, idx_k_w, q_down_w, q_up_w, q_rope_w,
           kv_down_w, kv_up_w, k_rope_w, out_w):
    x_mamba = _stage1(x, W_in, conv_w, conv_b, dt_bias, A_log, D_param,
                      norm_w, W_mamba_out)
    q_idx, k_idx, q_final, k_final, v = _stage2(
        x, x_mamba, idx_q_w, idx_k_w, q_down_w, q_up_w, q_rope_w,
        kv_down_w, kv_up_w, k_rope_w)
    return _stage3(q_idx, k_idx, q_final, k_final, v, out_w)


# speedup vs baseline: 21.8862x; 1.1184x over previous
import jax
import jax.numpy as jnp
from jax.experimental import pallas as pl
from jax.experimental.pallas import tpu as pltpu

SEQ = 2048
D_MODEL = 1024
D_STATE = 64
HEADDIM = 32
D_INNER = 1024
NHEADS_M = D_INNER // HEADDIM
CONV_K = 4
N_HEADS = 12
V_HEAD = 64
ROPE = 32
IDX_DIM = 64
TOP_K = 64

MAMBA_CHUNK = 128
PROJ_BLK = 512
ATTN_BLK = 256

NEG = float("-inf")


def _mm16(a, b, dims):
    return jax.lax.dot_general(
        a.astype(jnp.bfloat16), b.astype(jnp.bfloat16), (dims, ((), ())),
        preferred_element_type=jnp.float32)


def _mmd(a, b, dims):
    return jax.lax.dot_general(
        a, b, (dims, ((), ())), preferred_element_type=jnp.float32)


def _mamba_body(x_ref, w_in_ref, conv_w_ref, conv_b_ref, dt_bias_ref,
                a_log_ref, d_param_ref, norm_w_ref, w_out_ref,
                y_out_ref, h_ref, tail_ref, da_ref, xdt_ref, bc_ref, ys_ref):
    i = pl.program_id(0)

    @pl.when(i == 0)
    def _init():
        h_ref[...] = jnp.zeros_like(h_ref)
        tail_ref[...] = jnp.zeros_like(tail_ref)

    xc = x_ref[...]
    zxbcdt = _mmd(xc, w_in_ref[...], ((1,), (1,)))
    z = zxbcdt[:, :D_INNER]
    xbc_raw = zxbcdt[:, D_INNER:D_INNER + D_INNER + 2 * D_STATE]
    dt_raw = zxbcdt[:, -NHEADS_M:]

    padded = jnp.concatenate([tail_ref[...], xbc_raw], axis=0)
    acc = padded[0:MAMBA_CHUNK, :] * conv_w_ref[:, 0][None, :]
    for k in range(1, CONV_K):
        acc = acc + padded[k:k + MAMBA_CHUNK, :] * conv_w_ref[:, k][None, :]
    acc = acc + conv_b_ref[...]
    tail_ref[...] = xbc_raw[MAMBA_CHUNK - (CONV_K - 1):, :]
    xbc = acc * jax.nn.sigmoid(acc)

    bmat = xbc[:, D_INNER:D_INNER + D_STATE]
    cmat = xbc[:, D_INNER + D_STATE:]
    dt = jax.nn.softplus(dt_raw + dt_bias_ref[...])
    a_neg = -jnp.exp(a_log_ref[...])
    dloga = dt * a_neg

    q = MAMBA_CHUNK
    dA = jnp.exp(dloga)

    da_parts = []
    xdt_parts = []
    dxs_parts = []
    for h in range(NHEADS_M):
        xs_h = xbc[:, HEADDIM * h:HEADDIM * (h + 1)]
        da_parts.append(jnp.broadcast_to(dA[:, h:h + 1], (q, HEADDIM)))
        xdt_parts.append(dt[:, h:h + 1] * xs_h)
        dxs_parts.append(d_param_ref[0, h] * xs_h)
    da_ref[...] = jnp.concatenate(da_parts, axis=1)
    xdt_ref[...] = jnp.concatenate(xdt_parts, axis=1)
    dxs = jnp.concatenate(dxs_parts, axis=1)
    bc_ref[...] = jnp.concatenate([bmat, cmat], axis=1)

    def step(t, _):
        da_row = da_ref[pl.ds(t, 1), :]
        xdt_row = xdt_ref[pl.ds(t, 1), :]
        bc_row = bc_ref[pl.ds(t, 1), :]
        b_col = jnp.transpose(bc_row[:, :D_STATE])
        hst = h_ref[...] * da_row + xdt_row * b_col
        h_ref[...] = hst
        y_row = jax.lax.dot_general(
            bc_row[:, D_STATE:].astype(jnp.bfloat16),
            hst.astype(jnp.bfloat16),
            (((1,), (0,)), ((), ())),
            preferred_element_type=jnp.float32)
        ys_ref[pl.ds(t, 1), :] = y_row
        return 0

    jax.lax.fori_loop(0, q, step, 0, unroll=8)

    y = ys_ref[...] + dxs
    y = y * (z * jax.nn.sigmoid(z))
    y = y * jax.lax.rsqrt(jnp.mean(y * y, axis=-1, keepdims=True) + 1e-6)
    y = y * norm_w_ref[...]
    y_out_ref[...] = _mmd(y, w_out_ref[...], ((1,), (1,)))


def _rope2d(x, sin, cos):
    half = x.shape[-1] // 2
    rot = jnp.concatenate([-x[:, half:], x[:, :half]], axis=1)
    return x * cos + rot * sin


def _proj_body(x_ref, xm_ref, idx_q_ref, idx_k_ref, q_down_ref, q_up_ref,
               q_rope_ref, kv_down_ref, kv_up_ref, k_rope_ref, sin_ref, cos_ref,
               q_idx_out, k_idx_out, q_final_out, k_final_out, v_out):
    x = x_ref[...]
    xm = xm_ref[...]
    sin = sin_ref[...]
    cos = cos_ref[...]

    def mm_t(a, w):
        return _mm16(a, w, ((1,), (1,)))

    q_idx_out[...] = _mmd(xm, idx_q_ref[...], ((1,), (1,)))
    k_idx_out[...] = _mmd(x, idx_k_ref[...], ((1,), (1,)))

    c_q = mm_t(x, q_down_ref[...])
    q_content = mm_t(c_q, q_up_ref[...])
    q_rope = mm_t(c_q, q_rope_ref[...])
    c_kv = mm_t(x, kv_down_ref[...])
    kv = mm_t(c_kv, kv_up_ref[...])
    k_rope = _rope2d(mm_t(x, k_rope_ref[...]), sin, cos)

    q_parts = []
    k_parts = []
    for h in range(N_HEADS):
        qr_h = _rope2d(q_rope[:, ROPE * h:ROPE * (h + 1)], sin, cos)
        q_parts.append(q_content[:, V_HEAD * h:V_HEAD * (h + 1)])
        q_parts.append(qr_h)
        k_parts.append(kv[:, V_HEAD * h:V_HEAD * (h + 1)])
        k_parts.append(k_rope)
    q_final_out[...] = jnp.concatenate(q_parts, axis=1)
    k_final_out[...] = jnp.concatenate(k_parts, axis=1)
    v_out[...] = kv[:, N_HEADS * V_HEAD:]


def _attn_body(q_idx_ref, k_idx_ref, q_final_ref, k_final_ref, v_ref,
               out_w_ref, out_ref):
    b = pl.program_id(0)
    m = ATTN_BLK
    rowi = jax.lax.broadcasted_iota(jnp.int32, (m, SEQ), 0) + b * m
    coli = jax.lax.broadcasted_iota(jnp.int32, (m, SEQ), 1)
    causal = rowi >= coli

    s_idx = _mmd(q_idx_ref[...], k_idx_ref[...],
                 ((1,), (1,))) * (IDX_DIM ** -0.5)
    w0 = jnp.where(causal, s_idx, NEG)

    def ext_body(_, w):
        mx = jnp.max(w, axis=1, keepdims=True)
        return jnp.where(w == mx, NEG, w)

    w_fin = jax.lax.fori_loop(0, TOP_K, ext_body, w0)
    sel = ((rowi < TOP_K) & (coli < TOP_K)) | ((rowi >= TOP_K) & (w_fin != w0))

    scale = (V_HEAD + ROPE) ** -0.5
    head_outs = []
    for h in range(N_HEADS):
        hd = V_HEAD + ROPE
        q_h = q_final_ref[:, hd * h:hd * (h + 1)]
        k_h = k_final_ref[:, hd * h:hd * (h + 1)]
        v_h = v_ref[:, V_HEAD * h:V_HEAD * (h + 1)]
        logits = _mm16(q_h, k_h, ((1,), (1,))) * scale
        logits = jnp.where(sel, logits, NEG)
        mx = jnp.max(logits, axis=1, keepdims=True)
        p = jnp.exp(logits - mx)
        denom = jnp.sum(p, axis=1, keepdims=True)
        o_h = _mm16(p, v_h, ((1,), (0,)))
        head_outs.append(o_h / denom)
    attn = jnp.concatenate(head_outs, axis=1)
    out_ref[...] = _mm16(attn, out_w_ref[...], ((1,), (1,)))


def _stage1(x, W_in, conv_w, conv_b, dt_bias, A_log, D_param, norm_w,
            W_mamba_out):
    seq = x.shape[0]
    conv_dim = D_INNER + 2 * D_STATE
    d_in_proj = 2 * D_INNER + 2 * D_STATE + NHEADS_M

    conv_b2 = conv_b.reshape(1, conv_dim)
    dt_bias2 = dt_bias.reshape(1, NHEADS_M)
    a_log2 = A_log.reshape(1, NHEADS_M)
    d_param2 = D_param.reshape(1, NHEADS_M)
    norm_w2 = norm_w.reshape(1, D_INNER)

    n_chunks = seq // MAMBA_CHUNK
    x_mamba = pl.pallas_call(
        _mamba_body,
        grid=(n_chunks,),
        in_specs=[
            pl.BlockSpec((MAMBA_CHUNK, D_MODEL), lambda i: (i, 0)),
            pl.BlockSpec((d_in_proj, D_MODEL), lambda i: (0, 0)),
            pl.BlockSpec((conv_dim, CONV_K), lambda i: (0, 0)),
            pl.BlockSpec((1, conv_dim), lambda i: (0, 0)),
            pl.BlockSpec((1, NHEADS_M), lambda i: (0, 0)),
            pl.BlockSpec((1, NHEADS_M), lambda i: (0, 0)),
            pl.BlockSpec((1, NHEADS_M), lambda i: (0, 0)),
            pl.BlockSpec((1, D_INNER), lambda i: (0, 0)),
            pl.BlockSpec((D_MODEL, D_INNER), lambda i: (0, 0)),
        ],
        out_specs=pl.BlockSpec((MAMBA_CHUNK, D_MODEL), lambda i: (i, 0)),
        out_shape=jax.ShapeDtypeStruct((seq, D_MODEL), jnp.float32),
        scratch_shapes=[
            pltpu.VMEM((D_STATE, D_INNER), jnp.float32),
            pltpu.VMEM((CONV_K - 1, conv_dim), jnp.float32),
            pltpu.VMEM((MAMBA_CHUNK, D_INNER), jnp.float32),
            pltpu.VMEM((MAMBA_CHUNK, D_INNER), jnp.float32),
            pltpu.VMEM((MAMBA_CHUNK, 2 * D_STATE), jnp.float32),
            pltpu.VMEM((MAMBA_CHUNK, D_INNER), jnp.float32),
        ],
    )(x, W_in, conv_w, conv_b2, dt_bias2, a_log2, d_param2, norm_w2,
      W_mamba_out)
    return x_mamba


def _stage2(x, x_mamba, idx_q_w, idx_k_w, q_down_w, q_up_w, q_rope_w,
            kv_down_w, kv_up_w, k_rope_w):
    seq = x.shape[0]
    inv_freq = 1.0 / (10000.0 ** (jnp.arange(0, ROPE, 2, dtype=jnp.float32)
                                  / ROPE))
    t = jnp.arange(seq, dtype=jnp.float32)
    freqs = jnp.outer(t, inv_freq)
    emb = jnp.concatenate([freqs, freqs], axis=-1)
    sin, cos = jnp.sin(emb), jnp.cos(emb)

    n_pb = seq // PROJ_BLK
    q_idx, k_idx, q_final, k_final, v = pl.pallas_call(
        _proj_body,
        grid=(n_pb,),
        in_specs=[
            pl.BlockSpec((PROJ_BLK, D_MODEL), lambda i: (i, 0)),
            pl.BlockSpec((PROJ_BLK, D_MODEL), lambda i: (i, 0)),
            pl.BlockSpec((IDX_DIM, D_MODEL), lambda i: (0, 0)),
            pl.BlockSpec((IDX_DIM, D_MODEL), lambda i: (0, 0)),
            pl.BlockSpec((128, D_MODEL), lambda i: (0, 0)),
            pl.BlockSpec((N_HEADS * V_HEAD, 128), lambda i: (0, 0)),
            pl.BlockSpec((N_HEADS * ROPE, 128), lambda i: (0, 0)),
            pl.BlockSpec((128, D_MODEL), lambda i: (0, 0)),
            pl.BlockSpec((2 * N_HEADS * V_HEAD, 128), lambda i: (0, 0)),
            pl.BlockSpec((ROPE, D_MODEL), lambda i: (0, 0)),
            pl.BlockSpec((PROJ_BLK, ROPE), lambda i: (i, 0)),
            pl.BlockSpec((PROJ_BLK, ROPE), lambda i: (i, 0)),
        ],
        out_specs=[
            pl.BlockSpec((PROJ_BLK, IDX_DIM), lambda i: (i, 0)),
            pl.BlockSpec((PROJ_BLK, IDX_DIM), lambda i: (i, 0)),
            pl.BlockSpec((PROJ_BLK, N_HEADS * (V_HEAD + ROPE)),
                         lambda i: (i, 0)),
            pl.BlockSpec((PROJ_BLK, N_HEADS * (V_HEAD + ROPE)),
                         lambda i: (i, 0)),
            pl.BlockSpec((PROJ_BLK, N_HEADS * V_HEAD), lambda i: (i, 0)),
        ],
        out_shape=[
            jax.ShapeDtypeStruct((seq, IDX_DIM), jnp.float32),
            jax.ShapeDtypeStruct((seq, IDX_DIM), jnp.float32),
            jax.ShapeDtypeStruct((seq, N_HEADS * (V_HEAD + ROPE)), jnp.float32),
            jax.ShapeDtypeStruct((seq, N_HEADS * (V_HEAD + ROPE)), jnp.float32),
            jax.ShapeDtypeStruct((seq, N_HEADS * V_HEAD), jnp.float32),
        ],
    )(x, x_mamba, idx_q_w, idx_k_w, q_down_w, q_up_w, q_rope_w,
      kv_down_w, kv_up_w, k_rope_w, sin, cos)
    return q_idx, k_idx, q_final, k_final, v


def _stage3(q_idx, k_idx, q_final, k_final, v, out_w):
    seq = q_idx.shape[0]
    n_ab = seq // ATTN_BLK
    out = pl.pallas_call(
        _attn_body,
        grid=(n_ab,),
        in_specs=[
            pl.BlockSpec((ATTN_BLK, IDX_DIM), lambda i: (i, 0)),
            pl.BlockSpec((seq, IDX_DIM), lambda i: (0, 0)),
            pl.BlockSpec((ATTN_BLK, N_HEADS * (V_HEAD + ROPE)),
                         lambda i: (i, 0)),
            pl.BlockSpec((seq, N_HEADS * (V_HEAD + ROPE)), lambda i: (0, 0)),
            pl.BlockSpec((seq, N_HEADS * V_HEAD), lambda i: (0, 0)),
            pl.BlockSpec((D_MODEL, N_HEADS * V_HEAD), lambda i: (0, 0)),
        ],
        out_specs=pl.BlockSpec((ATTN_BLK, D_MODEL), lambda i: (i, 0)),
        out_shape=jax.ShapeDtypeStruct((seq, D_MODEL), jnp.float32),
    )(q_idx, k_idx, q_final, k_final, v, out_w)
    return out


def kernel(x, W_in, conv_w, conv_b, dt_bias, A_log, D_param, norm_w,
           W_mamba_out, idx_q_w, idx_k_w, q_down_w, q_up_w, q_rope_w,
           kv_down_w, kv_up_w, k_rope_w, out_w):
    x_mamba = _stage1(x, W_in, conv_w, conv_b, dt_bias, A_log, D_param,
                      norm_w, W_mamba_out)
    q_idx, k_idx, q_final, k_final, v = _stage2(
        x, x_mamba, idx_q_w, idx_k_w, q_down_w, q_up_w, q_rope_w,
        kv_down_w, kv_up_w, k_rope_w)
    return _stage3(q_idx, k_idx, q_final, k_final, v, out_w)


# speedup vs baseline: 22.9626x; 1.0492x over previous
import jax
import jax.numpy as jnp
from jax.experimental import pallas as pl
from jax.experimental.pallas import tpu as pltpu

SEQ = 2048
D_MODEL = 1024
D_STATE = 64
HEADDIM = 32
D_INNER = 1024
NHEADS_M = D_INNER // HEADDIM
CONV_K = 4
N_HEADS = 12
V_HEAD = 64
ROPE = 32
IDX_DIM = 64
TOP_K = 64

MAMBA_CHUNK = 128
PROJ_BLK = 512
ATTN_BLK = 256

NEG = float("-inf")


def _mm16(a, b, dims):
    return jax.lax.dot_general(
        a.astype(jnp.bfloat16), b.astype(jnp.bfloat16), (dims, ((), ())),
        preferred_element_type=jnp.float32)


def _mmd(a, b, dims):
    return jax.lax.dot_general(
        a, b, (dims, ((), ())), preferred_element_type=jnp.float32)


def _mamba_body(x_ref, w_in_ref, conv_w_ref, conv_b_ref, dt_bias_ref,
                a_log_ref, d_param_ref, norm_w_ref, w_out_ref,
                y_out_ref, h_ref, tail_ref, da_ref, xdt_ref, bc_ref, ys_ref):
    i = pl.program_id(0)

    @pl.when(i == 0)
    def _init():
        h_ref[...] = jnp.zeros_like(h_ref)
        tail_ref[...] = jnp.zeros_like(tail_ref)

    xc = x_ref[...]
    zxbcdt = _mmd(xc, w_in_ref[...], ((1,), (1,)))
    z = zxbcdt[:, :D_INNER]
    xbc_raw = zxbcdt[:, D_INNER:D_INNER + D_INNER + 2 * D_STATE]
    dt_raw = zxbcdt[:, -NHEADS_M:]

    padded = jnp.concatenate([tail_ref[...], xbc_raw], axis=0)
    acc = padded[0:MAMBA_CHUNK, :] * conv_w_ref[:, 0][None, :]
    for k in range(1, CONV_K):
        acc = acc + padded[k:k + MAMBA_CHUNK, :] * conv_w_ref[:, k][None, :]
    acc = acc + conv_b_ref[...]
    tail_ref[...] = xbc_raw[MAMBA_CHUNK - (CONV_K - 1):, :]
    xbc = acc * jax.nn.sigmoid(acc)

    bmat = xbc[:, D_INNER:D_INNER + D_STATE]
    cmat = xbc[:, D_INNER + D_STATE:]
    dt = jax.nn.softplus(dt_raw + dt_bias_ref[...])
    a_neg = -jnp.exp(a_log_ref[...])
    dloga = dt * a_neg

    q = MAMBA_CHUNK
    dA = jnp.exp(dloga)

    da_parts = []
    xdt_parts = []
    dxs_parts = []
    for h in range(NHEADS_M):
        xs_h = xbc[:, HEADDIM * h:HEADDIM * (h + 1)]
        da_parts.append(jnp.broadcast_to(dA[:, h:h + 1], (q, HEADDIM)))
        xdt_parts.append(dt[:, h:h + 1] * xs_h)
        dxs_parts.append(d_param_ref[0, h] * xs_h)
    da_ref[...] = jnp.concatenate(da_parts, axis=1)
    xdt_ref[...] = jnp.concatenate(xdt_parts, axis=1)
    dxs = jnp.concatenate(dxs_parts, axis=1)
    bc_ref[...] = jnp.concatenate([bmat, cmat], axis=1)

    def step(t, _):
        da_row = da_ref[pl.ds(t, 1), :]
        xdt_row = xdt_ref[pl.ds(t, 1), :]
        bc_row = bc_ref[pl.ds(t, 1), :]
        b_col = jnp.transpose(bc_row[:, :D_STATE])
        hst = h_ref[...] * da_row + xdt_row * b_col
        h_ref[...] = hst
        y_row = jax.lax.dot_general(
            bc_row[:, D_STATE:].astype(jnp.bfloat16),
            hst.astype(jnp.bfloat16),
            (((1,), (0,)), ((), ())),
            preferred_element_type=jnp.float32)
        ys_ref[pl.ds(t, 1), :] = y_row
        return 0

    jax.lax.fori_loop(0, q, step, 0, unroll=16)

    y = ys_ref[...] + dxs
    y = y * (z * jax.nn.sigmoid(z))
    y = y * jax.lax.rsqrt(jnp.mean(y * y, axis=-1, keepdims=True) + 1e-6)
    y = y * norm_w_ref[...]
    y_out_ref[...] = _mmd(y, w_out_ref[...], ((1,), (1,)))


def _rope2d(x, sin, cos):
    half = x.shape[-1] // 2
    rot = jnp.concatenate([-x[:, half:], x[:, :half]], axis=1)
    return x * cos + rot * sin


def _proj_body(x_ref, xm_ref, idx_q_ref, idx_k_ref, q_down_ref, q_up_ref,
               q_rope_ref, kv_down_ref, kv_up_ref, k_rope_ref, sin_ref, cos_ref,
               q_idx_out, k_idx_out, q_final_out, k_final_out, v_out):
    x = x_ref[...]
    xm = xm_ref[...]
    sin = sin_ref[...]
    cos = cos_ref[...]

    def mm_t(a, w):
        return _mm16(a, w, ((1,), (1,)))

    q_idx_out[...] = _mmd(xm, idx_q_ref[...], ((1,), (1,)))
    k_idx_out[...] = _mmd(x, idx_k_ref[...], ((1,), (1,)))

    c_q = mm_t(x, q_down_ref[...])
    q_content = mm_t(c_q, q_up_ref[...])
    q_rope = mm_t(c_q, q_rope_ref[...])
    c_kv = mm_t(x, kv_down_ref[...])
    kv = mm_t(c_kv, kv_up_ref[...])
    k_rope = _rope2d(mm_t(x, k_rope_ref[...]), sin, cos)

    q_parts = []
    k_parts = []
    for h in range(N_HEADS):
        qr_h = _rope2d(q_rope[:, ROPE * h:ROPE * (h + 1)], sin, cos)
        q_parts.append(q_content[:, V_HEAD * h:V_HEAD * (h + 1)])
        q_parts.append(qr_h)
        k_parts.append(kv[:, V_HEAD * h:V_HEAD * (h + 1)])
        k_parts.append(k_rope)
    q_final_out[...] = jnp.concatenate(q_parts, axis=1)
    k_final_out[...] = jnp.concatenate(k_parts, axis=1)
    v_out[...] = kv[:, N_HEADS * V_HEAD:]


def _attn_body(q_idx_ref, k_idx_ref, q_final_ref, k_final_ref, v_ref,
               out_w_ref, out_ref):
    b = pl.program_id(0)
    m = ATTN_BLK
    rowi = jax.lax.broadcasted_iota(jnp.int32, (m, SEQ), 0) + b * m
    coli = jax.lax.broadcasted_iota(jnp.int32, (m, SEQ), 1)
    causal = rowi >= coli

    s_idx = _mmd(q_idx_ref[...], k_idx_ref[...],
                 ((1,), (1,))) * (IDX_DIM ** -0.5)
    w0 = jnp.where(causal, s_idx, NEG)

    def ext_body(_, w):
        mx = jnp.max(w, axis=1, keepdims=True)
        return jnp.where(w == mx, NEG, w)

    w_fin = jax.lax.fori_loop(0, TOP_K, ext_body, w0)
    sel = ((rowi < TOP_K) & (coli < TOP_K)) | ((rowi >= TOP_K) & (w_fin != w0))

    scale = (V_HEAD + ROPE) ** -0.5
    head_outs = []
    for h in range(N_HEADS):
        hd = V_HEAD + ROPE
        q_h = q_final_ref[:, hd * h:hd * (h + 1)]
        k_h = k_final_ref[:, hd * h:hd * (h + 1)]
        v_h = v_ref[:, V_HEAD * h:V_HEAD * (h + 1)]
        logits = _mm16(q_h, k_h, ((1,), (1,))) * scale
        logits = jnp.where(sel, logits, NEG)
        mx = jnp.max(logits, axis=1, keepdims=True)
        p = jnp.exp(logits - mx)
        denom = jnp.sum(p, axis=1, keepdims=True)
        o_h = _mm16(p, v_h, ((1,), (0,)))
        head_outs.append(o_h / denom)
    attn = jnp.concatenate(head_outs, axis=1)
    out_ref[...] = _mm16(attn, out_w_ref[...], ((1,), (1,)))


def _stage1(x, W_in, conv_w, conv_b, dt_bias, A_log, D_param, norm_w,
            W_mamba_out):
    seq = x.shape[0]
    conv_dim = D_INNER + 2 * D_STATE
    d_in_proj = 2 * D_INNER + 2 * D_STATE + NHEADS_M

    conv_b2 = conv_b.reshape(1, conv_dim)
    dt_bias2 = dt_bias.reshape(1, NHEADS_M)
    a_log2 = A_log.reshape(1, NHEADS_M)
    d_param2 = D_param.reshape(1, NHEADS_M)
    norm_w2 = norm_w.reshape(1, D_INNER)

    n_chunks = seq // MAMBA_CHUNK
    x_mamba = pl.pallas_call(
        _mamba_body,
        grid=(n_chunks,),
        in_specs=[
            pl.BlockSpec((MAMBA_CHUNK, D_MODEL), lambda i: (i, 0)),
            pl.BlockSpec((d_in_proj, D_MODEL), lambda i: (0, 0)),
            pl.BlockSpec((conv_dim, CONV_K), lambda i: (0, 0)),
            pl.BlockSpec((1, conv_dim), lambda i: (0, 0)),
            pl.BlockSpec((1, NHEADS_M), lambda i: (0, 0)),
            pl.BlockSpec((1, NHEADS_M), lambda i: (0, 0)),
            pl.BlockSpec((1, NHEADS_M), lambda i: (0, 0)),
            pl.BlockSpec((1, D_INNER), lambda i: (0, 0)),
            pl.BlockSpec((D_MODEL, D_INNER), lambda i: (0, 0)),
        ],
        out_specs=pl.BlockSpec((MAMBA_CHUNK, D_MODEL), lambda i: (i, 0)),
        out_shape=jax.ShapeDtypeStruct((seq, D_MODEL), jnp.float32),
        scratch_shapes=[
            pltpu.VMEM((D_STATE, D_INNER), jnp.float32),
            pltpu.VMEM((CONV_K - 1, conv_dim), jnp.float32),
            pltpu.VMEM((MAMBA_CHUNK, D_INNER), jnp.float32),
            pltpu.VMEM((MAMBA_CHUNK, D_INNER), jnp.float32),
            pltpu.VMEM((MAMBA_CHUNK, 2 * D_STATE), jnp.float32),
            pltpu.VMEM((MAMBA_CHUNK, D_INNER), jnp.float32),
        ],
    )(x, W_in, conv_w, conv_b2, dt_bias2, a_log2, d_param2, norm_w2,
      W_mamba_out)
    return x_mamba


def _stage2(x, x_mamba, idx_q_w, idx_k_w, q_down_w, q_up_w, q_rope_w,
            kv_down_w, kv_up_w, k_rope_w):
    seq = x.shape[0]
    inv_freq = 1.0 / (10000.0 ** (jnp.arange(0, ROPE, 2, dtype=jnp.float32)
                                  / ROPE))
    t = jnp.arange(seq, dtype=jnp.float32)
    freqs = jnp.outer(t, inv_freq)
    emb = jnp.concatenate([freqs, freqs], axis=-1)
    sin, cos = jnp.sin(emb), jnp.cos(emb)

    n_pb = seq // PROJ_BLK
    q_idx, k_idx, q_final, k_final, v = pl.pallas_call(
        _proj_body,
        grid=(n_pb,),
        in_specs=[
            pl.BlockSpec((PROJ_BLK, D_MODEL), lambda i: (i, 0)),
            pl.BlockSpec((PROJ_BLK, D_MODEL), lambda i: (i, 0)),
            pl.BlockSpec((IDX_DIM, D_MODEL), lambda i: (0, 0)),
            pl.BlockSpec((IDX_DIM, D_MODEL), lambda i: (0, 0)),
            pl.BlockSpec((128, D_MODEL), lambda i: (0, 0)),
            pl.BlockSpec((N_HEADS * V_HEAD, 128), lambda i: (0, 0)),
            pl.BlockSpec((N_HEADS * ROPE, 128), lambda i: (0, 0)),
            pl.BlockSpec((128, D_MODEL), lambda i: (0, 0)),
            pl.BlockSpec((2 * N_HEADS * V_HEAD, 128), lambda i: (0, 0)),
            pl.BlockSpec((ROPE, D_MODEL), lambda i: (0, 0)),
            pl.BlockSpec((PROJ_BLK, ROPE), lambda i: (i, 0)),
            pl.BlockSpec((PROJ_BLK, ROPE), lambda i: (i, 0)),
        ],
        out_specs=[
            pl.BlockSpec((PROJ_BLK, IDX_DIM), lambda i: (i, 0)),
            pl.BlockSpec((PROJ_BLK, IDX_DIM), lambda i: (i, 0)),
            pl.BlockSpec((PROJ_BLK, N_HEADS * (V_HEAD + ROPE)),
                         lambda i: (i, 0)),
            pl.BlockSpec((PROJ_BLK, N_HEADS * (V_HEAD + ROPE)),
                         lambda i: (i, 0)),
            pl.BlockSpec((PROJ_BLK, N_HEADS * V_HEAD), lambda i: (i, 0)),
        ],
        out_shape=[
            jax.ShapeDtypeStruct((seq, IDX_DIM), jnp.float32),
            jax.ShapeDtypeStruct((seq, IDX_DIM), jnp.float32),
            jax.ShapeDtypeStruct((seq, N_HEADS * (V_HEAD + ROPE)), jnp.float32),
            jax.ShapeDtypeStruct((seq, N_HEADS * (V_HEAD + ROPE)), jnp.float32),
            jax.ShapeDtypeStruct((seq, N_HEADS * V_HEAD), jnp.float32),
        ],
    )(x, x_mamba, idx_q_w, idx_k_w, q_down_w, q_up_w, q_rope_w,
      kv_down_w, kv_up_w, k_rope_w, sin, cos)
    return q_idx, k_idx, q_final, k_final, v


def _stage3(q_idx, k_idx, q_final, k_final, v, out_w):
    seq = q_idx.shape[0]
    n_ab = seq // ATTN_BLK
    out = pl.pallas_call(
        _attn_body,
        grid=(n_ab,),
        in_specs=[
            pl.BlockSpec((ATTN_BLK, IDX_DIM), lambda i: (i, 0)),
            pl.BlockSpec((seq, IDX_DIM), lambda i: (0, 0)),
            pl.BlockSpec((ATTN_BLK, N_HEADS * (V_HEAD + ROPE)),
                         lambda i: (i, 0)),
            pl.BlockSpec((seq, N_HEADS * (V_HEAD + ROPE)), lambda i: (0, 0)),
            pl.BlockSpec((seq, N_HEADS * V_HEAD), lambda i: (0, 0)),
            pl.BlockSpec((D_MODEL, N_HEADS * V_HEAD), lambda i: (0, 0)),
        ],
        out_specs=pl.BlockSpec((ATTN_BLK, D_MODEL), lambda i: (i, 0)),
        out_shape=jax.ShapeDtypeStruct((seq, D_MODEL), jnp.float32),
    )(q_idx, k_idx, q_final, k_final, v, out_w)
    return out


def kernel(x, W_in, conv_w, conv_b, dt_bias, A_log, D_param, norm_w,
           W_mamba_out, idx_q_w, idx_k_w, q_down_w, q_up_w, q_rope_w,
           kv_down_w, kv_up_w, k_rope_w, out_w):
    x_mamba = _stage1(x, W_in, conv_w, conv_b, dt_bias, A_log, D_param,
                      norm_w, W_mamba_out)
    q_idx, k_idx, q_final, k_final, v = _stage2(
        x, x_mamba, idx_q_w, idx_k_w, q_down_w, q_up_w, q_rope_w,
        kv_down_w, kv_up_w, k_rope_w)
    return _stage3(q_idx, k_idx, q_final, k_final, v, out_w)
